# Initial kernel scaffold; baseline (speedup 1.0000x reference)
import jax
import jax.numpy as jnp
from jax.experimental import pallas as pl


def kernel(positions, node_attrs, shifts, atomic_energies, W_embed, W_up, W_r1, W_r2, W_down, W_sc, W_elem, W_p1, W_p2, W_ro, edge_index, batch, head, ptr):
    raise NotImplementedError("write your pallas kernel here")


# speedup vs baseline: 8.9803x; 8.9803x over previous
import functools
import math

import jax
import jax.numpy as jnp
from jax import lax
from jax.experimental import pallas as pl
from jax.experimental.pallas import tpu as pltpu

R_MAX = 5.0
H = 64
S = 4
NB = 8
EPS = 1e-9

NBLK = 1000
EBLK = 2000


def _silu(x):
    return x / (1.0 + jnp.exp(-x))


def _nodes_body(na_ref, b_ref, headf_ref, ae_ref, wemb_ref, wup0_ref,
                welem0_ref, welem1_ref,
                nf0_ref, h0_ref, ne0_ref, nh_ref, bg_ref, q0_ref, q1_ref,
                e0g_ref):
    i = pl.program_id(0)
    na = na_ref[...]
    b = b_ref[...]
    gids = lax.broadcasted_iota(jnp.int32, (na.shape[0], 64), 1)
    bg = (b == gids).astype(jnp.float32)
    nh = jnp.dot(bg, headf_ref[...], preferred_element_type=jnp.float32)
    ae = jnp.dot(na, ae_ref[...], preferred_element_type=jnp.float32)
    ne0 = ae[:, 0:1] * (1.0 - nh) + ae[:, 1:2] * nh
    nf0 = jnp.dot(na, wemb_ref[...], preferred_element_type=jnp.float32)
    nf0_ref[...] = nf0
    h0_ref[...] = jnp.dot(nf0, wup0_ref[...], preferred_element_type=jnp.float32)
    ne0_ref[...] = ne0
    nh_ref[...] = nh
    bg_ref[...] = bg
    q0_ref[...] = jnp.dot(na, welem0_ref[...], preferred_element_type=jnp.float32)
    q1_ref[...] = jnp.dot(na, welem1_ref[...], preferred_element_type=jnp.float32)
    part = lax.dot_general(bg, ne0, (((0,), (0,)), ((), ())),
                           preferred_element_type=jnp.float32)
    prev = jnp.where(i == 0, jnp.zeros_like(part), e0g_ref[...])
    e0g_ref[...] = prev + part


def _run_nodes(node_attrs, batch_col, head_f, atomic_energies, W_embed,
               W_up0, W_elem0, W_elem1):
    n = node_attrs.shape[0]
    ne = node_attrs.shape[1]
    grid = n // NBLK
    full = lambda shape: pl.BlockSpec(shape, lambda i: tuple(0 for _ in shape))
    row = lambda w: pl.BlockSpec((NBLK, w), lambda i: (i, 0))
    outs = (
        jax.ShapeDtypeStruct((n, H), jnp.float32),
        jax.ShapeDtypeStruct((n, H), jnp.float32),
        jax.ShapeDtypeStruct((n, 1), jnp.float32),
        jax.ShapeDtypeStruct((n, 1), jnp.float32),
        jax.ShapeDtypeStruct((n, 64), jnp.float32),
        jax.ShapeDtypeStruct((n, H), jnp.float32),
        jax.ShapeDtypeStruct((n, H), jnp.float32),
        jax.ShapeDtypeStruct((64, 1), jnp.float32),
    )
    return pl.pallas_call(
        _nodes_body,
        grid=(grid,),
        in_specs=[row(ne), row(1), full((64, 1)), full((ne, 2)),
                  full((ne, H)), full((H, H)), full((ne, H)), full((ne, H))],
        out_specs=(row(H), row(H), row(1), row(1), row(64), row(H), row(H),
                   full((64, 1))),
        out_shape=outs,
    )(node_attrs, batch_col, head_f, atomic_energies, W_embed, W_up0,
      W_elem0, W_elem1)


def _edges_body(ps_ref, pr_ref, sh_ref, ea_ref, ef_ref):
    ps = ps_ref[...]
    pr = pr_ref[...]
    d = pr[:, :3] - ps[:, :3] + sh_ref[...]
    l2 = jnp.sum(d * d, axis=1, keepdims=True)
    lg = jnp.sqrt(l2)
    vn = d / (lg + EPS)
    c0 = 0.28209479177387814
    c1 = 0.4886025119029199
    ones = jnp.full((d.shape[0], 1), c0, dtype=jnp.float32)
    ea_ref[...] = jnp.concatenate([ones, c1 * vn], axis=1)
    ns = lax.broadcasted_iota(jnp.int32, (1, NB), 1).astype(jnp.float32) + 1.0
    pref = math.sqrt(2.0 / R_MAX)
    rb = pref * jnp.sin(lg * (ns * (math.pi / R_MAX))) / (lg + EPS)
    u = lg / R_MAX
    u2 = u * u
    u3 = u2 * u
    u6 = u3 * u3
    u7 = u6 * u
    u8 = u6 * u2
    cut = 1.0 - 28.0 * u6 + 48.0 * u7 - 21.0 * u8
    cut = jnp.where(u < 1.0, cut, 0.0)
    ef_ref[...] = rb * cut


def _run_edges(ps_pad, pr_pad, shifts):
    e = ps_pad.shape[0]
    grid = e // EBLK
    row = lambda w: pl.BlockSpec((EBLK, w), lambda i: (i, 0))
    outs = (
        jax.ShapeDtypeStruct((e, S), jnp.float32),
        jax.ShapeDtypeStruct((e, NB), jnp.float32),
    )
    return pl.pallas_call(
        _edges_body,
        grid=(grid,),
        in_specs=[row(8), row(8), row(3)],
        out_specs=(row(S), row(NB)),
        out_shape=outs,
    )(ps_pad, pr_pad, shifts)


def _msg_body(ef_ref, ea_ref, hs_ref, wr1_ref, wr2p_ref, m0_ref, m1_ref):
    t = _silu(jnp.dot(ef_ref[...], wr1_ref[...],
                      preferred_element_type=jnp.float32))
    rs = jnp.dot(t, wr2p_ref[...], preferred_element_type=jnp.float32)
    ea = ea_ref[...]
    hs = hs_ref[...]
    hs2 = jnp.concatenate([hs, hs], axis=1)
    n = ea.shape[0]
    ea01 = jnp.concatenate(
        [jnp.broadcast_to(ea[:, 0:1], (n, H)),
         jnp.broadcast_to(ea[:, 1:2], (n, H))], axis=1)
    ea23 = jnp.concatenate(
        [jnp.broadcast_to(ea[:, 2:3], (n, H)),
         jnp.broadcast_to(ea[:, 3:4], (n, H))], axis=1)
    m0_ref[...] = rs[:, :128] * ea01 * hs2
    m1_ref[...] = rs[:, 128:] * ea23 * hs2


def _run_msg(ef, ea, hs, W_r1l, W_r2lp):
    e = ef.shape[0]
    grid = e // EBLK
    full = lambda shape: pl.BlockSpec(shape, lambda i: tuple(0 for _ in shape))
    row = lambda w: pl.BlockSpec((EBLK, w), lambda i: (i, 0))
    outs = (
        jax.ShapeDtypeStruct((e, 128), jnp.float32),
        jax.ShapeDtypeStruct((e, 128), jnp.float32),
    )
    return pl.pallas_call(
        _msg_body,
        grid=(grid,),
        in_specs=[row(NB), row(S), row(H), full((NB, H)), full((H, H * S))],
        out_specs=(row(128), row(128)),
        out_shape=outs,
    )(ef, ea, hs, W_r1l, W_r2lp)


def _upd_body(a0_ref, a1_ref, nf_ref, q_ref, nh_ref, bg_ref,
              wdp_ref, wsc_ref, wp1_ref, wp2_ref, wro_ref, wupn_ref,
              nf2_ref, hn_ref, ne_ref, eg_ref):
    i = pl.program_id(0)
    agg = jnp.concatenate([a0_ref[...], a1_ref[...]], axis=1)
    msg = jnp.dot(agg, wdp_ref[...], preferred_element_type=jnp.float32)
    nf = nf_ref[...]
    sc = jnp.dot(nf, wsc_ref[...], preferred_element_type=jnp.float32) * q_ref[...]
    nf2 = (jnp.dot(msg, wp1_ref[...], preferred_element_type=jnp.float32)
           + jnp.dot(msg * msg, wp2_ref[...], preferred_element_type=jnp.float32)
           + sc)
    nf2_ref[...] = nf2
    hn_ref[...] = jnp.dot(nf2, wupn_ref[...], preferred_element_type=jnp.float32)
    ro = jnp.dot(nf2, wro_ref[...], preferred_element_type=jnp.float32)
    nh = nh_ref[...]
    ne = ro[:, 0:1] * (1.0 - nh) + ro[:, 1:2] * nh
    ne_ref[...] = ne
    part = lax.dot_general(bg_ref[...], ne, (((0,), (0,)), ((), ())),
                           preferred_element_type=jnp.float32)
    prev = jnp.where(i == 0, jnp.zeros_like(part), eg_ref[...])
    eg_ref[...] = prev + part


def _run_update(agg0, agg1, nf, q, nh, bg, W_down_p, W_scl, W_p1l, W_p2l,
                W_rol, W_upn):
    n = nf.shape[0]
    grid = n // NBLK
    full = lambda shape: pl.BlockSpec(shape, lambda i: tuple(0 for _ in shape))
    row = lambda w: pl.BlockSpec((NBLK, w), lambda i: (i, 0))
    outs = (
        jax.ShapeDtypeStruct((n, H), jnp.float32),
        jax.ShapeDtypeStruct((n, H), jnp.float32),
        jax.ShapeDtypeStruct((n, 1), jnp.float32),
        jax.ShapeDtypeStruct((64, 1), jnp.float32),
    )
    return pl.pallas_call(
        _upd_body,
        grid=(grid,),
        in_specs=[row(128), row(128), row(H), row(H), row(1), row(64),
                  full((H * S, H)), full((H, H)), full((H, H)), full((H, H)),
                  full((H, 2)), full((H, H))],
        out_specs=(row(H), row(H), row(1), full((64, 1))),
        out_shape=outs,
    )(agg0, agg1, nf, q, nh, bg, W_down_p, W_scl, W_p1l, W_p2l, W_rol, W_upn)


def _gather_rows(table, idx):
    return jnp.take(table, idx, axis=0)


def _scatter_add(m0, m1, receiver, n):
    agg0 = jnp.zeros((n, 128), jnp.float32).at[receiver].add(m0)
    agg1 = jnp.zeros((n, 128), jnp.float32).at[receiver].add(m1)
    return agg0, agg1


def kernel(positions, node_attrs, shifts, atomic_energies, W_embed, W_up,
           W_r1, W_r2, W_down, W_sc, W_elem, W_p1, W_p2, W_ro, edge_index,
           batch, head, ptr):
    n = positions.shape[0]
    sender = edge_index[0]
    receiver = edge_index[1]

    perm = (jnp.arange(H * S) % H) * S + (jnp.arange(H * S) // H)
    W_r2p = W_r2[:, :, perm]
    W_down_p = W_down[:, perm, :]
    head_f = head.astype(jnp.float32).reshape(64, 1)
    batch_col = batch.astype(jnp.int32).reshape(n, 1)
    pos_pad = jnp.pad(positions, ((0, 0), (0, 5)))

    (nf, h, ne0, nh, bg, q0, q1, e0g) = _run_nodes(
        node_attrs, batch_col, head_f, atomic_energies, W_embed, W_up[0],
        W_elem[0], W_elem[1])

    ps = _gather_rows(pos_pad, sender)
    pr = _gather_rows(pos_pad, receiver)

    ea, ef = _run_edges(ps, pr, shifts)

    qs = (q0, q1)
    nf_list = []
    ne_list = []
    eg_list = []
    for l in range(2):
        hs = _gather_rows(h, sender)
        m0, m1 = _run_msg(ef, ea, hs, W_r1[l], W_r2p[l])
        agg0, agg1 = _scatter_add(m0, m1, receiver, n)
        wupn = W_up[1] if l == 0 else W_up[1]
        nf, h, ne, eg = _run_update(agg0, agg1, nf, qs[l], nh, bg,
                                    W_down_p[l], W_sc[l], W_p1[l], W_p2[l],
                                    W_ro[l], wupn)
        nf_list.append(nf)
        ne_list.append(ne)
        eg_list.append(eg)

    e0 = e0g[:, 0]
    e1 = eg_list[0][:, 0]
    e2 = eg_list[1][:, 0]
    zero_g = jnp.zeros_like(e0)
    total_energy = e0 + e1 + e2
    contributions = jnp.stack([e0, zero_g, e1, e2], axis=-1)
    node_energy = (ne0 + ne_list[0] + ne_list[1])[:, 0]
    node_feats_out = jnp.concatenate(nf_list, axis=-1)
    return total_energy, node_energy, contributions, node_feats_out


# speedup vs baseline: 28.2686x; 3.1478x over previous
import functools
import math

import jax
import jax.numpy as jnp
from jax import lax
from jax.experimental import pallas as pl
from jax.experimental.pallas import tpu as pltpu
from jax.experimental.pallas import tpu_sc as plsc

R_MAX = 5.0
H = 64
S = 4
NB = 8
EPS = 1e-9

NBLK = 1000
EBLK = 2000


def _silu(x):
    return x / (1.0 + jnp.exp(-x))


def _nodes_body(na_ref, b_ref, pos_ref, headf_ref, ae_ref, wemb_ref,
                wup0_ref, welem0_ref, welem1_ref,
                nf0_ref, t0_ref, ne0_ref, nh_ref, bg_ref, q0_ref, q1_ref,
                e0g_ref):
    i = pl.program_id(0)
    na = na_ref[...]
    b = b_ref[...]
    gids = lax.broadcasted_iota(jnp.int32, (na.shape[0], 64), 1)
    bg = (b == gids).astype(jnp.float32)
    nh = jnp.dot(bg, headf_ref[...], preferred_element_type=jnp.float32)
    ae = jnp.dot(na, ae_ref[...], preferred_element_type=jnp.float32)
    ne0 = ae[:, 0:1] * (1.0 - nh) + ae[:, 1:2] * nh
    nf0 = jnp.dot(na, wemb_ref[...], preferred_element_type=jnp.float32)
    nf0_ref[...] = nf0
    h0 = jnp.dot(nf0, wup0_ref[...], preferred_element_type=jnp.float32)
    zpad = jnp.zeros((na.shape[0], 64 - 3), jnp.float32)
    t0_ref[...] = jnp.concatenate([h0, pos_ref[...], zpad], axis=1)
    ne0_ref[...] = ne0
    nh_ref[...] = nh
    bg_ref[...] = bg
    q0_ref[...] = jnp.dot(na, welem0_ref[...], preferred_element_type=jnp.float32)
    q1_ref[...] = jnp.dot(na, welem1_ref[...], preferred_element_type=jnp.float32)
    part = lax.dot_general(bg, ne0, (((0,), (0,)), ((), ())),
                           preferred_element_type=jnp.float32)
    prev = jnp.where(i == 0, jnp.zeros_like(part), e0g_ref[...])
    e0g_ref[...] = prev + part


def _run_nodes(node_attrs, batch_col, positions, head_f, atomic_energies,
               W_embed, W_up0, W_elem0, W_elem1):
    n = node_attrs.shape[0]
    ne = node_attrs.shape[1]
    grid = n // NBLK
    full = lambda shape: pl.BlockSpec(shape, lambda i: tuple(0 for _ in shape))
    row = lambda w: pl.BlockSpec((NBLK, w), lambda i: (i, 0))
    outs = (
        jax.ShapeDtypeStruct((n, H), jnp.float32),
        jax.ShapeDtypeStruct((n, 128), jnp.float32),
        jax.ShapeDtypeStruct((n, 1), jnp.float32),
        jax.ShapeDtypeStruct((n, 1), jnp.float32),
        jax.ShapeDtypeStruct((n, 64), jnp.float32),
        jax.ShapeDtypeStruct((n, H), jnp.float32),
        jax.ShapeDtypeStruct((n, H), jnp.float32),
        jax.ShapeDtypeStruct((64, 1), jnp.float32),
    )
    return pl.pallas_call(
        _nodes_body,
        grid=(grid,),
        in_specs=[row(ne), row(1), row(3), full((64, 1)), full((ne, 2)),
                  full((ne, H)), full((H, H)), full((ne, H)), full((ne, H))],
        out_specs=(row(H), row(128), row(1), row(1), row(64), row(H), row(H),
                   full((64, 1))),
        out_shape=outs,
    )(node_attrs, batch_col, positions, head_f, atomic_energies, W_embed,
      W_up0, W_elem0, W_elem1)


def _edges_body(ps_ref, pr_ref, sh_ref, ea_ref, ef_ref):
    ps = ps_ref[...]
    pr = pr_ref[...]
    d = pr[:, 64:67] - ps[:, 64:67] + sh_ref[...]
    l2 = jnp.sum(d * d, axis=1, keepdims=True)
    lg = jnp.sqrt(l2)
    vn = d / (lg + EPS)
    c0 = 0.28209479177387814
    c1 = 0.4886025119029199
    ones = jnp.full((d.shape[0], 1), c0, dtype=jnp.float32)
    ea_ref[...] = jnp.concatenate([ones, c1 * vn], axis=1)
    ns = lax.broadcasted_iota(jnp.int32, (1, NB), 1).astype(jnp.float32) + 1.0
    pref = math.sqrt(2.0 / R_MAX)
    rb = pref * jnp.sin(lg * (ns * (math.pi / R_MAX))) / (lg + EPS)
    u = lg / R_MAX
    u2 = u * u
    u3 = u2 * u
    u6 = u3 * u3
    u7 = u6 * u
    u8 = u6 * u2
    cut = 1.0 - 28.0 * u6 + 48.0 * u7 - 21.0 * u8
    cut = jnp.where(u < 1.0, cut, 0.0)
    ef_ref[...] = rb * cut


def _run_edges(g_s, g_r, shifts):
    e = g_s.shape[0]
    grid = e // EBLK
    row = lambda w: pl.BlockSpec((EBLK, w), lambda i: (i, 0))
    outs = (
        jax.ShapeDtypeStruct((e, S), jnp.float32),
        jax.ShapeDtypeStruct((e, NB), jnp.float32),
    )
    return pl.pallas_call(
        _edges_body,
        grid=(grid,),
        in_specs=[row(128), row(128), row(3)],
        out_specs=(row(S), row(NB)),
        out_shape=outs,
    )(g_s, g_r, shifts)


def _msg_body(dup, ef_ref, ea_ref, hs_ref, wr1_ref, wr2p_ref, m_ref):
    t = _silu(jnp.dot(ef_ref[...], wr1_ref[...],
                      preferred_element_type=jnp.float32))
    rs = jnp.dot(t, wr2p_ref[...], preferred_element_type=jnp.float32)
    ea = ea_ref[...]
    x = hs_ref[...]
    if dup:
        hs2 = jnp.concatenate([x[:, :H], x[:, :H]], axis=1)
    else:
        hs2 = x
    n = ea.shape[0]
    ea01 = jnp.concatenate(
        [jnp.broadcast_to(ea[:, 0:1], (n, H)),
         jnp.broadcast_to(ea[:, 1:2], (n, H))], axis=1)
    ea23 = jnp.concatenate(
        [jnp.broadcast_to(ea[:, 2:3], (n, H)),
         jnp.broadcast_to(ea[:, 3:4], (n, H))], axis=1)
    m_ref[0] = rs[:, :128] * ea01 * hs2
    m_ref[1] = rs[:, 128:] * ea23 * hs2


def _run_msg(ef, ea, hs_full, W_r1l, W_r2lp, dup):
    e = ef.shape[0]
    grid = e // EBLK
    full = lambda shape: pl.BlockSpec(shape, lambda i: tuple(0 for _ in shape))
    row = lambda w: pl.BlockSpec((EBLK, w), lambda i: (i, 0))
    return pl.pallas_call(
        functools.partial(_msg_body, dup),
        grid=(grid,),
        in_specs=[row(NB), row(S), row(128), full((NB, H)), full((H, H * S))],
        out_specs=pl.BlockSpec((2, EBLK, 128), lambda i: (0, i, 0)),
        out_shape=jax.ShapeDtypeStruct((2, e, 128), jnp.float32),
    )(ef, ea, hs_full, W_r1l, W_r2lp)


def _upd_body(a_ref, nf_ref, q_ref, nh_ref, bg_ref,
              wdp_ref, wsc_ref, wp1_ref, wp2_ref, wro_ref, wupn_ref,
              nf2_ref, hn_ref, ne_ref, eg_ref):
    i = pl.program_id(0)
    agg = jnp.concatenate([a_ref[0], a_ref[1]], axis=1)
    msg = jnp.dot(agg, wdp_ref[...], preferred_element_type=jnp.float32)
    nf = nf_ref[...]
    sc = jnp.dot(nf, wsc_ref[...], preferred_element_type=jnp.float32) * q_ref[...]
    nf2 = (jnp.dot(msg, wp1_ref[...], preferred_element_type=jnp.float32)
           + jnp.dot(msg * msg, wp2_ref[...], preferred_element_type=jnp.float32)
           + sc)
    nf2_ref[...] = nf2
    hn = jnp.dot(nf2, wupn_ref[...], preferred_element_type=jnp.float32)
    hn_ref[...] = jnp.concatenate([hn, hn], axis=1)
    ro = jnp.dot(nf2, wro_ref[...], preferred_element_type=jnp.float32)
    nh = nh_ref[...]
    ne = ro[:, 0:1] * (1.0 - nh) + ro[:, 1:2] * nh
    ne_ref[...] = ne
    part = lax.dot_general(bg_ref[...], ne, (((0,), (0,)), ((), ())),
                           preferred_element_type=jnp.float32)
    prev = jnp.where(i == 0, jnp.zeros_like(part), eg_ref[...])
    eg_ref[...] = prev + part


def _run_update(agg, nf, q, nh, bg, W_down_p, W_scl, W_p1l, W_p2l,
                W_rol, W_upn):
    n = nf.shape[0]
    grid = n // NBLK
    full = lambda shape: pl.BlockSpec(shape, lambda i: tuple(0 for _ in shape))
    row = lambda w: pl.BlockSpec((NBLK, w), lambda i: (i, 0))
    outs = (
        jax.ShapeDtypeStruct((n, H), jnp.float32),
        jax.ShapeDtypeStruct((n, 128), jnp.float32),
        jax.ShapeDtypeStruct((n, 1), jnp.float32),
        jax.ShapeDtypeStruct((64, 1), jnp.float32),
    )
    return pl.pallas_call(
        _upd_body,
        grid=(grid,),
        in_specs=[pl.BlockSpec((2, NBLK, 128), lambda i: (0, i, 0)),
                  row(H), row(H), row(1), row(64),
                  full((H * S, H)), full((H, H)), full((H, H)), full((H, H)),
                  full((H, 2)), full((H, H))],
        out_specs=(row(H), row(128), row(1), full((64, 1))),
        out_shape=outs,
    )(agg, nf, q, nh, bg, W_down_p, W_scl, W_p1l, W_p2l, W_rol, W_upn)


SC_NC = 2
SC_NS = 16
SC_NW = SC_NC * SC_NS

N_NODES = 10000
N_PAD = 10240
NPW = N_PAD // SC_NS
N_EDGES = 320000
EPW = N_EDGES // SC_NS
KS = 80
NKS = EPW // KS
EPG = N_EDGES // SC_NW
KG = 80
NKG = EPG // KG


def _sc_mesh():
    return plsc.VectorSubcoreMesh(core_axis_name="c", subcore_axis_name="s",
                                  num_cores=SC_NC, num_subcores=SC_NS)


def _gather_body(tab_ref, ei_ref, out_ref, idx_v, rows_v, isem, gsem,
                 osem):
    wid = lax.axis_index("s") * SC_NC + lax.axis_index("c")
    base = wid * EPG

    def start_idx(k, slot):
        pltpu.async_copy(ei_ref.at[pl.ds(base + k * KG, KG)],
                         idx_v.at[slot], isem.at[slot])

    def start_gather(slot):
        pltpu.async_copy(tab_ref.at[idx_v.at[slot]], rows_v.at[slot],
                         gsem.at[slot])

    def start_out(k, slot):
        pltpu.async_copy(rows_v.at[slot],
                         out_ref.at[pl.ds(base + k * KG, KG)], osem.at[slot])

    def wait(sem, slot, src, dst):
        pltpu.make_async_copy(src, dst, sem.at[slot]).wait()

    start_idx(0, 0)
    wait(isem, 0, ei_ref.at[pl.ds(0, KG)], idx_v.at[0])
    start_gather(0)

    def body(k, carry):
        slot = lax.rem(k, 2)
        nslot = lax.rem(k + 1, 2)

        @pl.when(k + 1 < NKG)
        def _():
            @pl.when(k >= 1)
            def _():
                wait(osem, nslot, rows_v.at[0], out_ref.at[pl.ds(0, KG)])
            start_idx(k + 1, nslot)
            wait(isem, nslot, ei_ref.at[pl.ds(0, KG)], idx_v.at[nslot])
            start_gather(nslot)

        wait(gsem, slot, tab_ref.at[idx_v.at[slot]], rows_v.at[slot])
        start_out(k, slot)
        return carry

    lax.fori_loop(0, NKG, body, 0, unroll=2)
    wait(osem, lax.rem(NKG - 2, 2), rows_v.at[0], out_ref.at[pl.ds(0, KG)])
    wait(osem, lax.rem(NKG - 1, 2), rows_v.at[0], out_ref.at[pl.ds(0, KG)])


def _sc_gather(table, idx1d):
    d = table.shape[1]
    return pl.kernel(
        _gather_body,
        out_type=jax.ShapeDtypeStruct((N_EDGES, d), jnp.float32),
        mesh=_sc_mesh(),
        scratch_types=[
            pltpu.VMEM((2, KG), jnp.int32),
            pltpu.VMEM((2, KG, d), jnp.float32),
            pltpu.SemaphoreType.DMA((2,)),
            pltpu.SemaphoreType.DMA((2,)),
            pltpu.SemaphoreType.DMA((2,)),
        ],
    )(table, idx1d)


def _scatter_body(m_ref, ei_ref, zrow_ref, agg_ref,
                  agg_sh, idx_v, m_v, isem, msem, ssem):
    c = lax.axis_index("c")
    sid = lax.axis_index("s")
    base = sid * EPW

    pltpu.sync_copy(zrow_ref, agg_sh.at[pl.ds(sid * NPW, NPW)])
    plsc.subcore_barrier()

    def start_load(k, slot):
        off = base + k * KS
        pltpu.async_copy(ei_ref.at[pl.ds(off, KS)], idx_v.at[slot],
                         isem.at[slot])
        pltpu.async_copy(m_ref.at[c, pl.ds(off, KS)], m_v.at[slot],
                         msem.at[slot])

    def wait_load(slot):
        pltpu.make_async_copy(ei_ref.at[pl.ds(0, KS)], idx_v.at[slot],
                              isem.at[slot]).wait()
        pltpu.make_async_copy(m_ref.at[0, pl.ds(0, KS)], m_v.at[slot],
                              msem.at[slot]).wait()

    def start_scatter(slot):
        pltpu.async_copy(m_v.at[slot], agg_sh.at[idx_v.at[slot]],
                         ssem.at[slot], add=True)

    def wait_scatter(slot):
        pltpu.make_async_copy(m_v.at[slot], agg_sh.at[idx_v.at[slot]],
                              ssem.at[slot]).wait()

    start_load(0, 0)

    def body(k, carry):
        slot = lax.rem(k, 2)
        nslot = lax.rem(k + 1, 2)

        @pl.when(k + 1 < NKS)
        def _():
            @pl.when(k >= 1)
            def _():
                wait_scatter(nslot)
            start_load(k + 1, nslot)

        wait_load(slot)
        start_scatter(slot)
        return carry

    lax.fori_loop(0, NKS, body, 0, unroll=2)
    wait_scatter(lax.rem(NKS - 2, 2))
    wait_scatter(lax.rem(NKS - 1, 2))
    plsc.subcore_barrier()
    pltpu.sync_copy(agg_sh.at[pl.ds(sid * NPW, NPW)],
                    agg_ref.at[c, pl.ds(sid * NPW, NPW)])


def _sc_scatter(m, recv, zrow):
    return pl.kernel(
        _scatter_body,
        out_type=jax.ShapeDtypeStruct((2, N_PAD, 128), jnp.float32),
        mesh=_sc_mesh(),
        scratch_types=[
            pltpu.VMEM_SHARED((N_PAD, 128), jnp.float32),
            pltpu.VMEM((2, KS), jnp.int32),
            pltpu.VMEM((2, KS, 128), jnp.float32),
            pltpu.SemaphoreType.DMA((2,)),
            pltpu.SemaphoreType.DMA((2,)),
            pltpu.SemaphoreType.DMA((2,)),
        ],
    )(m, recv, zrow)


def kernel(positions, node_attrs, shifts, atomic_energies, W_embed, W_up,
           W_r1, W_r2, W_down, W_sc, W_elem, W_p1, W_p2, W_ro, edge_index,
           batch, head, ptr):
    n = positions.shape[0]
    sender = edge_index[0].astype(jnp.int32)
    receiver = edge_index[1].astype(jnp.int32)

    perm = (jnp.arange(H * S) % H) * S + (jnp.arange(H * S) // H)
    W_r2p = W_r2[:, :, perm]
    W_down_p = W_down[:, perm, :]
    head_f = head.astype(jnp.float32).reshape(64, 1)
    batch_col = batch.astype(jnp.int32).reshape(n, 1)

    (nf, t, ne0, nh, bg, q0, q1, e0g) = _run_nodes(
        node_attrs, batch_col, positions, head_f, atomic_energies, W_embed,
        W_up[0], W_elem[0], W_elem[1])

    g_s = _sc_gather(t, sender)
    g_r = _sc_gather(t, receiver)

    ea, ef = _run_edges(g_s, g_r, shifts)

    qs = (q0, q1)
    nf_list = []
    ne_list = []
    eg_list = []
    zrow = jnp.zeros((NPW, 128), jnp.float32)
    for l in range(2):
        if l > 0:
            g_s = _sc_gather(t, sender)
        m = _run_msg(ef, ea, g_s, W_r1[l], W_r2p[l], dup=(l == 0))
        agg = _sc_scatter(m, receiver, zrow)[:, :n]
        nf, t, ne, eg = _run_update(agg, nf, qs[l], nh, bg,
                                    W_down_p[l], W_sc[l], W_p1[l], W_p2[l],
                                    W_ro[l], W_up[1])
        nf_list.append(nf)
        ne_list.append(ne)
        eg_list.append(eg)

    e0 = e0g[:, 0]
    e1 = eg_list[0][:, 0]
    e2 = eg_list[1][:, 0]
    zero_g = jnp.zeros_like(e0)
    total_energy = e0 + e1 + e2
    contributions = jnp.stack([e0, zero_g, e1, e2], axis=-1)
    node_energy = (ne0 + ne_list[0] + ne_list[1])[:, 0]
    node_feats_out = jnp.concatenate(nf_list, axis=-1)
    return total_energy, node_energy, contributions, node_feats_out


# speedup vs baseline: 34.7043x; 1.2277x over previous
import functools
import math

import jax
import jax.numpy as jnp
from jax import lax
from jax.experimental import pallas as pl
from jax.experimental.pallas import tpu as pltpu
from jax.experimental.pallas import tpu_sc as plsc

R_MAX = 5.0
H = 64
S = 4
NB = 8
EPS = 1e-9

NBLK = 1000
EBLK = 2000


def _silu(x):
    return x / (1.0 + jnp.exp(-x))


def _nodes_body(na_ref, b_ref, pos_ref, headf_ref, ae_ref, wemb_ref,
                wup0_ref, welem0_ref, welem1_ref,
                nf0_ref, t0_ref, ne0_ref, nh_ref, bg_ref, q0_ref, q1_ref,
                e0g_ref):
    i = pl.program_id(0)
    na = na_ref[...]
    b = b_ref[...]
    gids = lax.broadcasted_iota(jnp.int32, (na.shape[0], 64), 1)
    bg = (b == gids).astype(jnp.float32)
    nh = jnp.dot(bg, headf_ref[...], preferred_element_type=jnp.float32)
    ae = jnp.dot(na, ae_ref[...], preferred_element_type=jnp.float32)
    ne0 = ae[:, 0:1] * (1.0 - nh) + ae[:, 1:2] * nh
    nf0 = jnp.dot(na, wemb_ref[...], preferred_element_type=jnp.float32)
    nf0_ref[...] = nf0
    h0 = jnp.dot(nf0, wup0_ref[...], preferred_element_type=jnp.float32)
    zpad = jnp.zeros((na.shape[0], 64 - 3), jnp.float32)
    t0_ref[...] = jnp.concatenate([h0, pos_ref[...], zpad], axis=1)
    ne0_ref[...] = ne0
    nh_ref[...] = nh
    bg_ref[...] = bg
    q0_ref[...] = jnp.dot(na, welem0_ref[...], preferred_element_type=jnp.float32)
    q1_ref[...] = jnp.dot(na, welem1_ref[...], preferred_element_type=jnp.float32)
    part = lax.dot_general(bg, ne0, (((0,), (0,)), ((), ())),
                           preferred_element_type=jnp.float32)
    prev = jnp.where(i == 0, jnp.zeros_like(part), e0g_ref[...])
    e0g_ref[...] = prev + part


def _run_nodes(node_attrs, batch_col, positions, head_f, atomic_energies,
               W_embed, W_up0, W_elem0, W_elem1):
    n = node_attrs.shape[0]
    ne = node_attrs.shape[1]
    grid = n // NBLK
    full = lambda shape: pl.BlockSpec(shape, lambda i: tuple(0 for _ in shape))
    row = lambda w: pl.BlockSpec((NBLK, w), lambda i: (i, 0))
    outs = (
        jax.ShapeDtypeStruct((n, H), jnp.float32),
        jax.ShapeDtypeStruct((n, 128), jnp.float32),
        jax.ShapeDtypeStruct((n, 1), jnp.float32),
        jax.ShapeDtypeStruct((n, 1), jnp.float32),
        jax.ShapeDtypeStruct((n, 64), jnp.float32),
        jax.ShapeDtypeStruct((n, H), jnp.float32),
        jax.ShapeDtypeStruct((n, H), jnp.float32),
        jax.ShapeDtypeStruct((64, 1), jnp.float32),
    )
    return pl.pallas_call(
        _nodes_body,
        grid=(grid,),
        in_specs=[row(ne), row(1), row(3), full((64, 1)), full((ne, 2)),
                  full((ne, H)), full((H, H)), full((ne, H)), full((ne, H))],
        out_specs=(row(H), row(128), row(1), row(1), row(64), row(H), row(H),
                   full((64, 1))),
        out_shape=outs,
    )(node_attrs, batch_col, positions, head_f, atomic_energies, W_embed,
      W_up0, W_elem0, W_elem1)


def _edges_body(ps_ref, pr_ref, sh_ref, ea_ref, lg_ref):
    ps = ps_ref[...]
    pr = pr_ref[...]
    d = pr[:, 64:67] - ps[:, 64:67] + sh_ref[...]
    l2 = jnp.sum(d * d, axis=1, keepdims=True)
    lg = jnp.sqrt(l2)
    vn = d / (lg + EPS)
    c0 = 0.28209479177387814
    c1 = 0.4886025119029199
    ones = jnp.full((d.shape[0], 1), c0, dtype=jnp.float32)
    ea_ref[...] = jnp.concatenate([ones, c1 * vn], axis=1)
    lg_ref[...] = lg


def _run_edges(g_s, g_r, shifts):
    e = g_s.shape[0]
    grid = e // EBLK
    row = lambda w: pl.BlockSpec((EBLK, w), lambda i: (i, 0))
    outs = (
        jax.ShapeDtypeStruct((e, S), jnp.float32),
        jax.ShapeDtypeStruct((e, 1), jnp.float32),
    )
    return pl.pallas_call(
        _edges_body,
        grid=(grid,),
        in_specs=[row(128), row(128), row(3)],
        out_specs=(row(S), row(1)),
        out_shape=outs,
    )(g_s, g_r, shifts)


RBLK = 500


def _radial_body(lg_ref, ef_ref):
    lg = lg_ref[...]
    x = jnp.minimum(lg, R_MAX) * (math.pi / R_MAX)
    t = x - (math.pi / 2.0)
    t2 = t * t
    s1 = 1.0 + t2 * (-0.5 + t2 * (1.0 / 24 + t2 * (-1.0 / 720 + t2 * (
        1.0 / 40320 + t2 * (-1.0 / 3628800 + t2 * (1.0 / 479001600))))))
    st = t * (1.0 + t2 * (-1.0 / 6 + t2 * (1.0 / 120 + t2 * (-1.0 / 5040
        + t2 * (1.0 / 362880 + t2 * (-1.0 / 39916800))))))
    two_c = -2.0 * st
    u = lg * (1.0 / R_MAX)
    u2 = u * u
    u3 = u2 * u
    u6 = u3 * u3
    cut = 1.0 - 28.0 * u6 + 48.0 * u6 * u - 21.0 * u6 * u2
    cut = jnp.where(u < 1.0, cut, 0.0)
    scale = math.sqrt(2.0 / R_MAX) * cut / (lg + EPS)
    s_nm1 = jnp.zeros_like(s1)
    s_n = s1
    for n in range(NB):
        ef_ref[n] = s_n * scale
        s_nm1, s_n = s_n, two_c * s_n - s_nm1


def _run_radial(lg):
    e = lg.shape[0]
    rows = e // 128
    lgd = jnp.reshape(lg, (rows, 128))
    planes = pl.pallas_call(
        _radial_body,
        out_shape=jax.ShapeDtypeStruct((NB, rows, 128), jnp.float32),
    )(lgd)
    return jnp.reshape(jnp.transpose(planes, (1, 2, 0)), (e, NB))


def _msg_body(dup, ef_ref, ea_ref, hs_ref, wr1_ref, wr2p_ref, sel_ref, m_ref):
    t = _silu(jnp.dot(ef_ref[...], wr1_ref[...],
                      preferred_element_type=jnp.float32))
    rs = jnp.dot(t, wr2p_ref[...], preferred_element_type=jnp.float32)
    ea = ea_ref[...]
    x = hs_ref[...]
    if dup:
        hs2 = jnp.concatenate([x[:, :H], x[:, :H]], axis=1)
    else:
        hs2 = x
    eab = jnp.dot(ea, sel_ref[...], preferred_element_type=jnp.float32)
    m_ref[0] = rs[:, :128] * eab[:, :128] * hs2
    m_ref[1] = rs[:, 128:] * eab[:, 128:] * hs2


def _run_msg(ef, ea, hs_full, W_r1l, W_r2lp, sel, dup):
    e = ef.shape[0]
    grid = e // EBLK
    full = lambda shape: pl.BlockSpec(shape, lambda i: tuple(0 for _ in shape))
    row = lambda w: pl.BlockSpec((EBLK, w), lambda i: (i, 0))
    return pl.pallas_call(
        functools.partial(_msg_body, dup),
        grid=(grid,),
        in_specs=[row(NB), row(S), row(128), full((NB, H)), full((H, H * S)),
                  full((S, H * S))],
        out_specs=pl.BlockSpec((2, EBLK, 128), lambda i: (0, i, 0)),
        out_shape=jax.ShapeDtypeStruct((2, e, 128), jnp.float32),
    )(ef, ea, hs_full, W_r1l, W_r2lp, sel)


def _upd_body(a_ref, nf_ref, q_ref, nh_ref, bg_ref,
              wdp_ref, wsc_ref, wp1_ref, wp2_ref, wro_ref, wupn_ref,
              nf2_ref, hn_ref, ne_ref, eg_ref):
    i = pl.program_id(0)
    agg = jnp.concatenate([a_ref[0], a_ref[1]], axis=1)
    msg = jnp.dot(agg, wdp_ref[...], preferred_element_type=jnp.float32)
    nf = nf_ref[...]
    sc = jnp.dot(nf, wsc_ref[...], preferred_element_type=jnp.float32) * q_ref[...]
    nf2 = (jnp.dot(msg, wp1_ref[...], preferred_element_type=jnp.float32)
           + jnp.dot(msg * msg, wp2_ref[...], preferred_element_type=jnp.float32)
           + sc)
    nf2_ref[...] = nf2
    hn = jnp.dot(nf2, wupn_ref[...], preferred_element_type=jnp.float32)
    hn_ref[...] = jnp.concatenate([hn, hn], axis=1)
    ro = jnp.dot(nf2, wro_ref[...], preferred_element_type=jnp.float32)
    nh = nh_ref[...]
    ne = ro[:, 0:1] * (1.0 - nh) + ro[:, 1:2] * nh
    ne_ref[...] = ne
    part = lax.dot_general(bg_ref[...], ne, (((0,), (0,)), ((), ())),
                           preferred_element_type=jnp.float32)
    prev = jnp.where(i == 0, jnp.zeros_like(part), eg_ref[...])
    eg_ref[...] = prev + part


def _run_update(agg, nf, q, nh, bg, W_down_p, W_scl, W_p1l, W_p2l,
                W_rol, W_upn):
    n = nf.shape[0]
    grid = n // NBLK
    full = lambda shape: pl.BlockSpec(shape, lambda i: tuple(0 for _ in shape))
    row = lambda w: pl.BlockSpec((NBLK, w), lambda i: (i, 0))
    outs = (
        jax.ShapeDtypeStruct((n, H), jnp.float32),
        jax.ShapeDtypeStruct((n, 128), jnp.float32),
        jax.ShapeDtypeStruct((n, 1), jnp.float32),
        jax.ShapeDtypeStruct((64, 1), jnp.float32),
    )
    return pl.pallas_call(
        _upd_body,
        grid=(grid,),
        in_specs=[pl.BlockSpec((2, NBLK, 128), lambda i: (0, i, 0)),
                  row(H), row(H), row(1), row(64),
                  full((H * S, H)), full((H, H)), full((H, H)), full((H, H)),
                  full((H, 2)), full((H, H))],
        out_specs=(row(H), row(128), row(1), full((64, 1))),
        out_shape=outs,
    )(agg, nf, q, nh, bg, W_down_p, W_scl, W_p1l, W_p2l, W_rol, W_upn)


SC_NC = 2
SC_NS = 16
SC_NW = SC_NC * SC_NS

N_NODES = 10000
N_PAD = 10240
NPW = N_PAD // SC_NS
N_EDGES = 320000
EPW = N_EDGES // SC_NS
KS = 80
NKS = EPW // KS
EPG = N_EDGES // SC_NW
KG = 80
NKG = EPG // KG


def _sc_mesh():
    return plsc.VectorSubcoreMesh(core_axis_name="c", subcore_axis_name="s",
                                  num_cores=SC_NC, num_subcores=SC_NS)


def _gather_body(tab_ref, ei_ref, out_ref, idx_v, rows_v, isem, gsem,
                 osem):
    wid = lax.axis_index("s") * SC_NC + lax.axis_index("c")
    base = wid * EPG

    def start_idx(k, slot):
        pltpu.async_copy(ei_ref.at[pl.ds(base + k * KG, KG)],
                         idx_v.at[slot], isem.at[slot])

    def start_gather(slot):
        pltpu.async_copy(tab_ref.at[idx_v.at[slot]], rows_v.at[slot],
                         gsem.at[slot])

    def start_out(k, slot):
        pltpu.async_copy(rows_v.at[slot],
                         out_ref.at[pl.ds(base + k * KG, KG)], osem.at[slot])

    def wait(sem, slot, src, dst):
        pltpu.make_async_copy(src, dst, sem.at[slot]).wait()

    start_idx(0, 0)
    wait(isem, 0, ei_ref.at[pl.ds(0, KG)], idx_v.at[0])
    start_gather(0)

    def body(k, carry):
        slot = lax.rem(k, 2)
        nslot = lax.rem(k + 1, 2)

        @pl.when(k + 1 < NKG)
        def _():
            @pl.when(k >= 1)
            def _():
                wait(osem, nslot, rows_v.at[0], out_ref.at[pl.ds(0, KG)])
            start_idx(k + 1, nslot)
            wait(isem, nslot, ei_ref.at[pl.ds(0, KG)], idx_v.at[nslot])
            start_gather(nslot)

        wait(gsem, slot, tab_ref.at[idx_v.at[slot]], rows_v.at[slot])
        start_out(k, slot)
        return carry

    lax.fori_loop(0, NKG, body, 0, unroll=2)
    wait(osem, lax.rem(NKG - 2, 2), rows_v.at[0], out_ref.at[pl.ds(0, KG)])
    wait(osem, lax.rem(NKG - 1, 2), rows_v.at[0], out_ref.at[pl.ds(0, KG)])


def _sc_gather(table, idx1d):
    d = table.shape[1]
    return pl.kernel(
        _gather_body,
        out_type=jax.ShapeDtypeStruct((N_EDGES, d), jnp.float32),
        mesh=_sc_mesh(),
        scratch_types=[
            pltpu.VMEM((2, KG), jnp.int32),
            pltpu.VMEM((2, KG, d), jnp.float32),
            pltpu.SemaphoreType.DMA((2,)),
            pltpu.SemaphoreType.DMA((2,)),
            pltpu.SemaphoreType.DMA((2,)),
        ],
    )(table, idx1d)


def _scatter_body(m_ref, ei_ref, zrow_ref, agg_ref,
                  agg_sh, idx_v, m_v, isem, msem, ssem):
    c = lax.axis_index("c")
    sid = lax.axis_index("s")
    base = sid * EPW

    pltpu.sync_copy(zrow_ref, agg_sh.at[pl.ds(sid * NPW, NPW)])
    plsc.subcore_barrier()

    def start_load(k, slot):
        off = base + k * KS
        pltpu.async_copy(ei_ref.at[pl.ds(off, KS)], idx_v.at[slot],
                         isem.at[slot])
        pltpu.async_copy(m_ref.at[c, pl.ds(off, KS)], m_v.at[slot],
                         msem.at[slot])

    def wait_load(slot):
        pltpu.make_async_copy(ei_ref.at[pl.ds(0, KS)], idx_v.at[slot],
                              isem.at[slot]).wait()
        pltpu.make_async_copy(m_ref.at[0, pl.ds(0, KS)], m_v.at[slot],
                              msem.at[slot]).wait()

    def start_scatter(slot):
        pltpu.async_copy(m_v.at[slot], agg_sh.at[idx_v.at[slot]],
                         ssem.at[slot], add=True)

    def wait_scatter(slot):
        pltpu.make_async_copy(m_v.at[slot], agg_sh.at[idx_v.at[slot]],
                              ssem.at[slot]).wait()

    start_load(0, 0)

    def body(k, carry):
        slot = lax.rem(k, 2)
        nslot = lax.rem(k + 1, 2)

        @pl.when(k + 1 < NKS)
        def _():
            @pl.when(k >= 1)
            def _():
                wait_scatter(nslot)
            start_load(k + 1, nslot)

        wait_load(slot)
        start_scatter(slot)
        return carry

    lax.fori_loop(0, NKS, body, 0, unroll=2)
    wait_scatter(lax.rem(NKS - 2, 2))
    wait_scatter(lax.rem(NKS - 1, 2))
    plsc.subcore_barrier()
    pltpu.sync_copy(agg_sh.at[pl.ds(sid * NPW, NPW)],
                    agg_ref.at[c, pl.ds(sid * NPW, NPW)])


def _sc_scatter(m, recv, zrow):
    return pl.kernel(
        _scatter_body,
        out_type=jax.ShapeDtypeStruct((2, N_PAD, 128), jnp.float32),
        mesh=_sc_mesh(),
        scratch_types=[
            pltpu.VMEM_SHARED((N_PAD, 128), jnp.float32),
            pltpu.VMEM((2, KS), jnp.int32),
            pltpu.VMEM((2, KS, 128), jnp.float32),
            pltpu.SemaphoreType.DMA((2,)),
            pltpu.SemaphoreType.DMA((2,)),
            pltpu.SemaphoreType.DMA((2,)),
        ],
    )(m, recv, zrow)


def kernel(positions, node_attrs, shifts, atomic_energies, W_embed, W_up,
           W_r1, W_r2, W_down, W_sc, W_elem, W_p1, W_p2, W_ro, edge_index,
           batch, head, ptr):
    n = positions.shape[0]
    sender = edge_index[0].astype(jnp.int32)
    receiver = edge_index[1].astype(jnp.int32)

    perm = (jnp.arange(H * S) % H) * S + (jnp.arange(H * S) // H)
    W_r2p = W_r2[:, :, perm]
    W_down_p = W_down[:, perm, :]
    head_f = head.astype(jnp.float32).reshape(64, 1)
    batch_col = batch.astype(jnp.int32).reshape(n, 1)

    (nf, t, ne0, nh, bg, q0, q1, e0g) = _run_nodes(
        node_attrs, batch_col, positions, head_f, atomic_energies, W_embed,
        W_up[0], W_elem[0], W_elem[1])

    g_s = _sc_gather(t, sender)
    g_r = _sc_gather(t, receiver)

    ea, lg = _run_edges(g_s, g_r, shifts)
    ef = _run_radial(lg)
    sel = (jnp.arange(H * S)[None, :] // H
           == jnp.arange(S)[:, None]).astype(jnp.float32)

    qs = (q0, q1)
    nf_list = []
    ne_list = []
    eg_list = []
    zrow = jnp.zeros((NPW, 128), jnp.float32)
    for l in range(2):
        if l > 0:
            g_s = _sc_gather(t, sender)
        m = _run_msg(ef, ea, g_s, W_r1[l], W_r2p[l], sel, dup=(l == 0))
        agg = _sc_scatter(m, receiver, zrow)[:, :n]
        nf, t, ne, eg = _run_update(agg, nf, qs[l], nh, bg,
                                    W_down_p[l], W_sc[l], W_p1[l], W_p2[l],
                                    W_ro[l], W_up[1])
        nf_list.append(nf)
        ne_list.append(ne)
        eg_list.append(eg)

    e0 = e0g[:, 0]
    e1 = eg_list[0][:, 0]
    e2 = eg_list[1][:, 0]
    zero_g = jnp.zeros_like(e0)
    total_energy = e0 + e1 + e2
    contributions = jnp.stack([e0, zero_g, e1, e2], axis=-1)
    node_energy = (ne0 + ne_list[0] + ne_list[1])[:, 0]
    node_feats_out = jnp.concatenate(nf_list, axis=-1)
    return total_energy, node_energy, contributions, node_feats_out


# speedup vs baseline: 37.3766x; 1.0770x over previous
import functools
import math

import jax
import jax.numpy as jnp
from jax import lax
from jax.experimental import pallas as pl
from jax.experimental.pallas import tpu as pltpu
from jax.experimental.pallas import tpu_sc as plsc

R_MAX = 5.0
H = 64
S = 4
NB = 8
EPS = 1e-9

NBLK = 1000
EBLK = 8000


def _silu(x):
    return x / (1.0 + jnp.exp(-x))


def _nodes_body(na_ref, b_ref, pos_ref, headf_ref, ae_ref, wemb_ref,
                wup0_ref, welem0_ref, welem1_ref,
                nf0_ref, t0_ref, ne0_ref, nh_ref, bg_ref, q0_ref, q1_ref,
                e0g_ref):
    i = pl.program_id(0)
    na = na_ref[...]
    b = b_ref[...]
    gids = lax.broadcasted_iota(jnp.int32, (na.shape[0], 64), 1)
    bg = (b == gids).astype(jnp.float32)
    nh = jnp.dot(bg, headf_ref[...], preferred_element_type=jnp.float32)
    ae = jnp.dot(na, ae_ref[...], preferred_element_type=jnp.float32)
    ne0 = ae[:, 0:1] * (1.0 - nh) + ae[:, 1:2] * nh
    nf0 = jnp.dot(na, wemb_ref[...], preferred_element_type=jnp.float32)
    nf0_ref[...] = nf0
    h0 = jnp.dot(nf0, wup0_ref[...], preferred_element_type=jnp.float32)
    zpad = jnp.zeros((na.shape[0], 64 - 3), jnp.float32)
    t0_ref[...] = jnp.concatenate([h0, pos_ref[...], zpad], axis=1)
    ne0_ref[...] = ne0
    nh_ref[...] = nh
    bg_ref[...] = bg
    q0_ref[...] = jnp.dot(na, welem0_ref[...], preferred_element_type=jnp.float32)
    q1_ref[...] = jnp.dot(na, welem1_ref[...], preferred_element_type=jnp.float32)
    part = lax.dot_general(bg, ne0, (((0,), (0,)), ((), ())),
                           preferred_element_type=jnp.float32)
    prev = jnp.where(i == 0, jnp.zeros_like(part), e0g_ref[...])
    e0g_ref[...] = prev + part


def _run_nodes(node_attrs, batch_col, positions, head_f, atomic_energies,
               W_embed, W_up0, W_elem0, W_elem1):
    n = node_attrs.shape[0]
    ne = node_attrs.shape[1]
    grid = n // NBLK
    full = lambda shape: pl.BlockSpec(shape, lambda i: tuple(0 for _ in shape))
    row = lambda w: pl.BlockSpec((NBLK, w), lambda i: (i, 0))
    outs = (
        jax.ShapeDtypeStruct((n, H), jnp.float32),
        jax.ShapeDtypeStruct((n, 128), jnp.float32),
        jax.ShapeDtypeStruct((n, 1), jnp.float32),
        jax.ShapeDtypeStruct((n, 1), jnp.float32),
        jax.ShapeDtypeStruct((n, 64), jnp.float32),
        jax.ShapeDtypeStruct((n, H), jnp.float32),
        jax.ShapeDtypeStruct((n, H), jnp.float32),
        jax.ShapeDtypeStruct((64, 1), jnp.float32),
    )
    return pl.pallas_call(
        _nodes_body,
        grid=(grid,),
        in_specs=[row(ne), row(1), row(3), full((64, 1)), full((ne, 2)),
                  full((ne, H)), full((H, H)), full((ne, H)), full((ne, H))],
        out_specs=(row(H), row(128), row(1), row(1), row(64), row(H), row(H),
                   full((64, 1))),
        out_shape=outs,
    )(node_attrs, batch_col, positions, head_f, atomic_energies, W_embed,
      W_up0, W_elem0, W_elem1)


def _edges_body(ps_ref, pr_ref, sh_ref, ea_ref, lg_ref):
    ps = ps_ref[...]
    pr = pr_ref[...]
    d = pr[:, 64:67] - ps[:, 64:67] + sh_ref[...]
    l2 = jnp.sum(d * d, axis=1, keepdims=True)
    lg = jnp.sqrt(l2)
    vn = d / (lg + EPS)
    c0 = 0.28209479177387814
    c1 = 0.4886025119029199
    ones = jnp.full((d.shape[0], 1), c0, dtype=jnp.float32)
    ea_ref[...] = jnp.concatenate([ones, c1 * vn], axis=1)
    lg_ref[...] = lg


def _run_edges(g_s, g_r, shifts):
    e = g_s.shape[0]
    grid = e // EBLK
    row = lambda w: pl.BlockSpec((EBLK, w), lambda i: (i, 0))
    outs = (
        jax.ShapeDtypeStruct((e, S), jnp.float32),
        jax.ShapeDtypeStruct((e, 1), jnp.float32),
    )
    return pl.pallas_call(
        _edges_body,
        grid=(grid,),
        in_specs=[row(128), row(128), row(3)],
        out_specs=(row(S), row(1)),
        out_shape=outs,
    )(g_s, g_r, shifts)


RBLK = 500


def _radial_body(lg_ref, ef_ref):
    lg = lg_ref[...]
    x = jnp.minimum(lg, R_MAX) * (math.pi / R_MAX)
    t = x - (math.pi / 2.0)
    t2 = t * t
    s1 = 1.0 + t2 * (-0.5 + t2 * (1.0 / 24 + t2 * (-1.0 / 720 + t2 * (
        1.0 / 40320 + t2 * (-1.0 / 3628800 + t2 * (1.0 / 479001600))))))
    st = t * (1.0 + t2 * (-1.0 / 6 + t2 * (1.0 / 120 + t2 * (-1.0 / 5040
        + t2 * (1.0 / 362880 + t2 * (-1.0 / 39916800))))))
    two_c = -2.0 * st
    u = lg * (1.0 / R_MAX)
    u2 = u * u
    u3 = u2 * u
    u6 = u3 * u3
    cut = 1.0 - 28.0 * u6 + 48.0 * u6 * u - 21.0 * u6 * u2
    cut = jnp.where(u < 1.0, cut, 0.0)
    scale = math.sqrt(2.0 / R_MAX) * cut / (lg + EPS)
    s_nm1 = jnp.zeros_like(s1)
    s_n = s1
    for n in range(NB):
        ef_ref[n] = s_n * scale
        s_nm1, s_n = s_n, two_c * s_n - s_nm1


def _run_radial(lg):
    e = lg.shape[0]
    rows = e // 128
    lgd = jnp.reshape(lg, (rows, 128))
    planes = pl.pallas_call(
        _radial_body,
        out_shape=jax.ShapeDtypeStruct((NB, rows, 128), jnp.float32),
    )(lgd)
    return jnp.reshape(jnp.transpose(planes, (1, 2, 0)), (e, NB))


def _msg_body(dup, ef_ref, ea_ref, hs_ref, wr1_ref, wr2p_ref, sel_ref, m_ref):
    t = _silu(jnp.dot(ef_ref[...], wr1_ref[...],
                      preferred_element_type=jnp.float32))
    rs = jnp.dot(t, wr2p_ref[...], preferred_element_type=jnp.float32)
    ea = ea_ref[...]
    x = hs_ref[...]
    if dup:
        hs2 = jnp.concatenate([x[:, :H], x[:, :H]], axis=1)
    else:
        hs2 = x
    eab = jnp.dot(ea, sel_ref[...], preferred_element_type=jnp.float32)
    m_ref[0] = rs[:, :128] * eab[:, :128] * hs2
    m_ref[1] = rs[:, 128:] * eab[:, 128:] * hs2


def _run_msg(ef, ea, hs_full, W_r1l, W_r2lp, sel, dup):
    e = ef.shape[0]
    grid = e // EBLK
    full = lambda shape: pl.BlockSpec(shape, lambda i: tuple(0 for _ in shape))
    row = lambda w: pl.BlockSpec((EBLK, w), lambda i: (i, 0))
    return pl.pallas_call(
        functools.partial(_msg_body, dup),
        grid=(grid,),
        in_specs=[row(NB), row(S), row(128), full((NB, H)), full((H, H * S)),
                  full((S, H * S))],
        out_specs=pl.BlockSpec((2, EBLK, 128), lambda i: (0, i, 0)),
        out_shape=jax.ShapeDtypeStruct((2, e, 128), jnp.float32),
    )(ef, ea, hs_full, W_r1l, W_r2lp, sel)


def _upd_body(a_ref, nf_ref, q_ref, nh_ref, bg_ref,
              wdp_ref, wsc_ref, wp1_ref, wp2_ref, wro_ref, wupn_ref,
              nf2_ref, hn_ref, ne_ref, eg_ref):
    i = pl.program_id(0)
    agg = jnp.concatenate([a_ref[0], a_ref[1]], axis=1)
    msg = jnp.dot(agg, wdp_ref[...], preferred_element_type=jnp.float32)
    nf = nf_ref[...]
    sc = jnp.dot(nf, wsc_ref[...], preferred_element_type=jnp.float32) * q_ref[...]
    nf2 = (jnp.dot(msg, wp1_ref[...], preferred_element_type=jnp.float32)
           + jnp.dot(msg * msg, wp2_ref[...], preferred_element_type=jnp.float32)
           + sc)
    nf2_ref[...] = nf2
    hn = jnp.dot(nf2, wupn_ref[...], preferred_element_type=jnp.float32)
    hn_ref[...] = jnp.concatenate([hn, hn], axis=1)
    ro = jnp.dot(nf2, wro_ref[...], preferred_element_type=jnp.float32)
    nh = nh_ref[...]
    ne = ro[:, 0:1] * (1.0 - nh) + ro[:, 1:2] * nh
    ne_ref[...] = ne
    part = lax.dot_general(bg_ref[...], ne, (((0,), (0,)), ((), ())),
                           preferred_element_type=jnp.float32)
    prev = jnp.where(i == 0, jnp.zeros_like(part), eg_ref[...])
    eg_ref[...] = prev + part


def _run_update(agg, nf, q, nh, bg, W_down_p, W_scl, W_p1l, W_p2l,
                W_rol, W_upn):
    n = nf.shape[0]
    grid = n // NBLK
    full = lambda shape: pl.BlockSpec(shape, lambda i: tuple(0 for _ in shape))
    row = lambda w: pl.BlockSpec((NBLK, w), lambda i: (i, 0))
    outs = (
        jax.ShapeDtypeStruct((n, H), jnp.float32),
        jax.ShapeDtypeStruct((n, 128), jnp.float32),
        jax.ShapeDtypeStruct((n, 1), jnp.float32),
        jax.ShapeDtypeStruct((64, 1), jnp.float32),
    )
    return pl.pallas_call(
        _upd_body,
        grid=(grid,),
        in_specs=[pl.BlockSpec((2, NBLK, 128), lambda i: (0, i, 0)),
                  row(H), row(H), row(1), row(64),
                  full((H * S, H)), full((H, H)), full((H, H)), full((H, H)),
                  full((H, 2)), full((H, H))],
        out_specs=(row(H), row(128), row(1), full((64, 1))),
        out_shape=outs,
    )(agg, nf, q, nh, bg, W_down_p, W_scl, W_p1l, W_p2l, W_rol, W_upn)


SC_NC = 2
SC_NS = 16
SC_NW = SC_NC * SC_NS

N_NODES = 10000
N_PAD = 10240
NPW = N_PAD // SC_NS
N_EDGES = 320000
EPW = N_EDGES // SC_NS
KS = 80
NKS = EPW // KS
EPG = N_EDGES // SC_NW
KG = 80
NKG = EPG // KG


def _sc_mesh():
    return plsc.VectorSubcoreMesh(core_axis_name="c", subcore_axis_name="s",
                                  num_cores=SC_NC, num_subcores=SC_NS)


def _gather_body(tab_ref, ei_ref, out_ref, idx_v, rows_v, isem, gsem,
                 osem):
    wid = lax.axis_index("s") * SC_NC + lax.axis_index("c")
    base = wid * EPG

    def start_idx(k, slot):
        pltpu.async_copy(ei_ref.at[pl.ds(base + k * KG, KG)],
                         idx_v.at[slot], isem.at[slot])

    def start_gather(slot):
        pltpu.async_copy(tab_ref.at[idx_v.at[slot]], rows_v.at[slot],
                         gsem.at[slot])

    def start_out(k, slot):
        pltpu.async_copy(rows_v.at[slot],
                         out_ref.at[pl.ds(base + k * KG, KG)], osem.at[slot])

    def wait(sem, slot, src, dst):
        pltpu.make_async_copy(src, dst, sem.at[slot]).wait()

    start_idx(0, 0)
    wait(isem, 0, ei_ref.at[pl.ds(0, KG)], idx_v.at[0])
    start_gather(0)

    def body(k, carry):
        slot = lax.rem(k, 2)
        nslot = lax.rem(k + 1, 2)

        @pl.when(k + 1 < NKG)
        def _():
            @pl.when(k >= 1)
            def _():
                wait(osem, nslot, rows_v.at[0], out_ref.at[pl.ds(0, KG)])
            start_idx(k + 1, nslot)
            wait(isem, nslot, ei_ref.at[pl.ds(0, KG)], idx_v.at[nslot])
            start_gather(nslot)

        wait(gsem, slot, tab_ref.at[idx_v.at[slot]], rows_v.at[slot])
        start_out(k, slot)
        return carry

    lax.fori_loop(0, NKG, body, 0, unroll=2)
    wait(osem, lax.rem(NKG - 2, 2), rows_v.at[0], out_ref.at[pl.ds(0, KG)])
    wait(osem, lax.rem(NKG - 1, 2), rows_v.at[0], out_ref.at[pl.ds(0, KG)])


def _sc_gather(table, idx1d):
    d = table.shape[1]
    return pl.kernel(
        _gather_body,
        out_type=jax.ShapeDtypeStruct((N_EDGES, d), jnp.float32),
        mesh=_sc_mesh(),
        scratch_types=[
            pltpu.VMEM((2, KG), jnp.int32),
            pltpu.VMEM((2, KG, d), jnp.float32),
            pltpu.SemaphoreType.DMA((2,)),
            pltpu.SemaphoreType.DMA((2,)),
            pltpu.SemaphoreType.DMA((2,)),
        ],
    )(table, idx1d)


def _scatter_body(m_ref, ei_ref, zrow_ref, agg_ref,
                  agg_sh, idx_v, m_v, isem, msem, ssem):
    c = lax.axis_index("c")
    sid = lax.axis_index("s")
    base = sid * EPW

    pltpu.sync_copy(zrow_ref, agg_sh.at[pl.ds(sid * NPW, NPW)])
    plsc.subcore_barrier()

    def start_load(k, slot):
        off = base + k * KS
        pltpu.async_copy(ei_ref.at[pl.ds(off, KS)], idx_v.at[slot],
                         isem.at[slot])
        pltpu.async_copy(m_ref.at[c, pl.ds(off, KS)], m_v.at[slot],
                         msem.at[slot])

    def wait_load(slot):
        pltpu.make_async_copy(ei_ref.at[pl.ds(0, KS)], idx_v.at[slot],
                              isem.at[slot]).wait()
        pltpu.make_async_copy(m_ref.at[0, pl.ds(0, KS)], m_v.at[slot],
                              msem.at[slot]).wait()

    def start_scatter(slot):
        pltpu.async_copy(m_v.at[slot], agg_sh.at[idx_v.at[slot]],
                         ssem.at[slot], add=True)

    def wait_scatter(slot):
        pltpu.make_async_copy(m_v.at[slot], agg_sh.at[idx_v.at[slot]],
                              ssem.at[slot]).wait()

    start_load(0, 0)

    def body(k, carry):
        slot = lax.rem(k, 2)
        nslot = lax.rem(k + 1, 2)

        @pl.when(k + 1 < NKS)
        def _():
            @pl.when(k >= 1)
            def _():
                wait_scatter(nslot)
            start_load(k + 1, nslot)

        wait_load(slot)
        start_scatter(slot)
        return carry

    lax.fori_loop(0, NKS, body, 0, unroll=2)
    wait_scatter(lax.rem(NKS - 2, 2))
    wait_scatter(lax.rem(NKS - 1, 2))
    plsc.subcore_barrier()
    pltpu.sync_copy(agg_sh.at[pl.ds(sid * NPW, NPW)],
                    agg_ref.at[c, pl.ds(sid * NPW, NPW)])


def _sc_scatter(m, recv, zrow):
    return pl.kernel(
        _scatter_body,
        out_type=jax.ShapeDtypeStruct((2, N_PAD, 128), jnp.float32),
        mesh=_sc_mesh(),
        scratch_types=[
            pltpu.VMEM_SHARED((N_PAD, 128), jnp.float32),
            pltpu.VMEM((2, KS), jnp.int32),
            pltpu.VMEM((2, KS, 128), jnp.float32),
            pltpu.SemaphoreType.DMA((2,)),
            pltpu.SemaphoreType.DMA((2,)),
            pltpu.SemaphoreType.DMA((2,)),
        ],
    )(m, recv, zrow)


def kernel(positions, node_attrs, shifts, atomic_energies, W_embed, W_up,
           W_r1, W_r2, W_down, W_sc, W_elem, W_p1, W_p2, W_ro, edge_index,
           batch, head, ptr):
    n = positions.shape[0]
    sender = edge_index[0].astype(jnp.int32)
    receiver = edge_index[1].astype(jnp.int32)

    perm = (jnp.arange(H * S) % H) * S + (jnp.arange(H * S) // H)
    W_r2p = W_r2[:, :, perm]
    W_down_p = W_down[:, perm, :]
    head_f = head.astype(jnp.float32).reshape(64, 1)
    batch_col = batch.astype(jnp.int32).reshape(n, 1)

    (nf, t, ne0, nh, bg, q0, q1, e0g) = _run_nodes(
        node_attrs, batch_col, positions, head_f, atomic_energies, W_embed,
        W_up[0], W_elem[0], W_elem[1])

    g_s = _sc_gather(t, sender)
    g_r = _sc_gather(t, receiver)

    ea, lg = _run_edges(g_s, g_r, shifts)
    ef = _run_radial(lg)
    sel = (jnp.arange(H * S)[None, :] // H
           == jnp.arange(S)[:, None]).astype(jnp.float32)

    qs = (q0, q1)
    nf_list = []
    ne_list = []
    eg_list = []
    zrow = jnp.zeros((NPW, 128), jnp.float32)
    for l in range(2):
        if l > 0:
            g_s = _sc_gather(t, sender)
        m = _run_msg(ef, ea, g_s, W_r1[l], W_r2p[l], sel, dup=(l == 0))
        agg = _sc_scatter(m, receiver, zrow)[:, :n]
        nf, t, ne, eg = _run_update(agg, nf, qs[l], nh, bg,
                                    W_down_p[l], W_sc[l], W_p1[l], W_p2[l],
                                    W_ro[l], W_up[1])
        nf_list.append(nf)
        ne_list.append(ne)
        eg_list.append(eg)

    e0 = e0g[:, 0]
    e1 = eg_list[0][:, 0]
    e2 = eg_list[1][:, 0]
    zero_g = jnp.zeros_like(e0)
    total_energy = e0 + e1 + e2
    contributions = jnp.stack([e0, zero_g, e1, e2], axis=-1)
    node_energy = (ne0 + ne_list[0] + ne_list[1])[:, 0]
    node_feats_out = jnp.concatenate(nf_list, axis=-1)
    return total_energy, node_energy, contributions, node_feats_out
